# baseline (speedup 1.0000x reference)
import functools

import jax
import jax.numpy as jnp
from jax import lax
from jax.experimental import pallas as pl
from jax.experimental.pallas import tpu as pltpu
from jax.experimental.pallas import tpu_sc as plsc

N_NODES = 10000
N_EDGES = 320000
D = 128

NC = 2
NS = 16
NW = NC * NS

CHUNK = 80
NBUF = 4
NGROUP = 32
NCHUNK = NGROUP * NBUF
EPT = NCHUNK * CHUNK
E_PAD = EPT * NW
SINK = N_NODES
AGG_ROWS = 10112
RPT = AGG_ROWS // NS
OPT = 632
OPT_LAST = N_NODES - 15 * OPT

_mesh = plsc.VectorSubcoreMesh(core_axis_name="c", subcore_axis_name="s")


@functools.partial(
    pl.kernel,
    mesh=_mesh,
    out_type=jax.ShapeDtypeStruct((NC, N_NODES, D), jnp.float32),
    scratch_types=[
        pltpu.VMEM((2, NBUF, CHUNK), jnp.int32),
        pltpu.VMEM((2, NBUF, CHUNK), jnp.int32),
        pltpu.VMEM((NBUF, CHUNK, D), jnp.float32),
        pltpu.VMEM_SHARED((AGG_ROWS, D), jnp.float32),
        pltpu.SemaphoreType.DMA((NBUF,)),
        pltpu.SemaphoreType.DMA((NBUF,)),
        pltpu.SemaphoreType.DMA((2,)),
        pltpu.SemaphoreType.DMA((2,)),
    ],
)
def _sc_agg(x_hbm, src_hbm, dst_hbm, zeros_hbm, out_hbm,
            sidx, didx, rows_v, agg_s, gsem, ssem, fsem_s, fsem_d):
    c = lax.axis_index("c")
    s = lax.axis_index("s")
    wid = c * NS + s

    pltpu.sync_copy(zeros_hbm.at[pl.ds(s * RPT, RPT)],
                    agg_s.at[pl.ds(s * RPT, RPT)])
    plsc.subcore_barrier()

    def gather(p, b):
        pltpu.async_copy(x_hbm.at[sidx.at[p, b]], rows_v.at[b], gsem.at[b])

    def gather_wait(p, b):
        pltpu.make_async_copy(x_hbm.at[sidx.at[p, b]], rows_v.at[b],
                              gsem.at[b]).wait()

    def scatter_start(p, b):
        pltpu.async_copy(rows_v.at[b], agg_s.at[didx.at[p, b]], ssem.at[b],
                         add=True)

    def scatter_wait(p, b):
        pltpu.make_async_copy(rows_v.at[b], agg_s.at[didx.at[p, b]],
                              ssem.at[b]).wait()

    pltpu.sync_copy(src_hbm.at[wid, 0], sidx.at[0])
    pltpu.sync_copy(dst_hbm.at[wid, 0], didx.at[0])
    for b in range(NBUF):
        gather(0, b)

    def group_body(g, _):
        p = lax.rem(g, 2)
        q = 1 - p
        have_next = g + 1 < NGROUP

        @pl.when(have_next)
        def _():
            pltpu.async_copy(src_hbm.at[wid, g + 1], sidx.at[q], fsem_s.at[q])
            pltpu.async_copy(dst_hbm.at[wid, g + 1], didx.at[q], fsem_d.at[q])

        for b in range(NBUF):
            gather_wait(p, b)
            scatter_start(p, b)

        @pl.when(have_next)
        def _():
            pltpu.make_async_copy(src_hbm.at[wid, g + 1], sidx.at[q],
                                  fsem_s.at[q]).wait()
            pltpu.make_async_copy(dst_hbm.at[wid, g + 1], didx.at[q],
                                  fsem_d.at[q]).wait()

        for b in range(NBUF):
            scatter_wait(p, b)

            @pl.when(have_next)
            def _():
                gather(q, b)

        return 0

    lax.fori_loop(0, NGROUP, group_body, 0)
    plsc.subcore_barrier()

    @pl.when(s < NS - 1)
    def _():
        pltpu.sync_copy(agg_s.at[pl.ds(s * OPT, OPT)],
                        out_hbm.at[c, pl.ds(s * OPT, OPT)])

    @pl.when(s == NS - 1)
    def _():
        pltpu.sync_copy(agg_s.at[pl.ds(15 * OPT, OPT_LAST)],
                        out_hbm.at[c, pl.ds(15 * OPT, OPT_LAST)])


BLK = 1000


def _tc_body(x_ref, a0_ref, a1_ref, w1_ref, b1_ref, w2_ref, b2_ref,
             wl_ref, bl_ref, o_ref):
    xb = x_ref[...]
    h = xb + a0_ref[...] + a1_ref[...]
    h = jnp.dot(h, w1_ref[...], preferred_element_type=jnp.float32) + b1_ref[...]
    h = jnp.maximum(h, 0.0)
    h = jnp.dot(h, w2_ref[...], preferred_element_type=jnp.float32) + b2_ref[...]
    res = jnp.dot(xb, wl_ref[...], preferred_element_type=jnp.float32) + bl_ref[...]
    o_ref[...] = res + h


def _tc_mlp(x, a0, a1, W1, b1, W2, b2, Wl, bl):
    grid = (N_NODES // BLK,)
    row_spec = pl.BlockSpec((BLK, D), lambda i: (i, 0))
    w_spec = pl.BlockSpec((D, D), lambda i: (0, 0))
    b_spec = pl.BlockSpec((1, D), lambda i: (0, 0))
    return pl.pallas_call(
        _tc_body,
        grid=grid,
        in_specs=[row_spec, row_spec, row_spec,
                  w_spec, b_spec, w_spec, b_spec, w_spec, b_spec],
        out_specs=row_spec,
        out_shape=jax.ShapeDtypeStruct((N_NODES, D), jnp.float32),
    )(x, a0, a1, W1, b1, W2, b2, Wl, bl)


@jax.jit
def kernel(x, edge_index, W1, b1, W2, b2, Wl, bl):
    ei = edge_index.astype(jnp.int32)
    pad = E_PAD - N_EDGES
    src = jnp.concatenate([ei[0], jnp.zeros((pad,), jnp.int32)])
    sink_rows = SINK + jnp.arange(pad, dtype=jnp.int32) % (AGG_ROWS - SINK)
    dst = jnp.concatenate([ei[1], sink_rows])
    src = src.reshape(NW, NGROUP, NBUF, CHUNK)
    dst = dst.reshape(NW, NGROUP, NBUF, CHUNK)
    zeros = jnp.zeros((AGG_ROWS, D), jnp.float32)
    agg = _sc_agg(x, src, dst, zeros)
    return _tc_mlp(x, agg[0], agg[1], W1,
                   b1.reshape(1, D), W2, b2.reshape(1, D),
                   Wl, bl.reshape(1, D))


# speedup vs baseline: 1.0007x; 1.0007x over previous
import functools

import jax
import jax.numpy as jnp
from jax import lax
from jax.experimental import pallas as pl
from jax.experimental.pallas import tpu as pltpu
from jax.experimental.pallas import tpu_sc as plsc

N_NODES = 10000
N_EDGES = 320000
D = 128

NC = 2
NS = 16
NW = NC * NS

CHUNK = 128
NBUF = 2
NGROUP = 40
NCHUNK = NGROUP * NBUF
EPT = NCHUNK * CHUNK
E_PAD = EPT * NW
SINK = N_NODES
AGG_ROWS = 10112
RPT = AGG_ROWS // NS
OPT = 632
OPT_LAST = N_NODES - 15 * OPT

_mesh = plsc.VectorSubcoreMesh(core_axis_name="c", subcore_axis_name="s")


@functools.partial(
    pl.kernel,
    mesh=_mesh,
    out_type=jax.ShapeDtypeStruct((NC, N_NODES, D), jnp.float32),
    scratch_types=[
        pltpu.VMEM((2, NBUF, CHUNK), jnp.int32),
        pltpu.VMEM((2, NBUF, CHUNK), jnp.int32),
        pltpu.VMEM((NBUF, CHUNK, D), jnp.float32),
        pltpu.VMEM_SHARED((AGG_ROWS, D), jnp.float32),
        pltpu.SemaphoreType.DMA((NBUF,)),
        pltpu.SemaphoreType.DMA((NBUF,)),
        pltpu.SemaphoreType.DMA((2,)),
        pltpu.SemaphoreType.DMA((2,)),
    ],
)
def _sc_agg(x_hbm, src_hbm, dst_hbm, zeros_hbm, out_hbm,
            sidx, didx, rows_v, agg_s, gsem, ssem, fsem_s, fsem_d):
    c = lax.axis_index("c")
    s = lax.axis_index("s")
    wid = c * NS + s

    pltpu.sync_copy(zeros_hbm.at[pl.ds(s * RPT, RPT)],
                    agg_s.at[pl.ds(s * RPT, RPT)])
    plsc.subcore_barrier()

    def gather(p, b):
        pltpu.async_copy(x_hbm.at[sidx.at[p, b]], rows_v.at[b], gsem.at[b])

    def gather_wait(p, b):
        pltpu.make_async_copy(x_hbm.at[sidx.at[p, b]], rows_v.at[b],
                              gsem.at[b]).wait()

    def scatter_start(p, b):
        pltpu.async_copy(rows_v.at[b], agg_s.at[didx.at[p, b]], ssem.at[b],
                         add=True)

    def scatter_wait(p, b):
        pltpu.make_async_copy(rows_v.at[b], agg_s.at[didx.at[p, b]],
                              ssem.at[b]).wait()

    pltpu.sync_copy(src_hbm.at[wid, 0], sidx.at[0])
    pltpu.sync_copy(dst_hbm.at[wid, 0], didx.at[0])
    for b in range(NBUF):
        gather(0, b)

    def group_body(g, _):
        p = lax.rem(g, 2)
        q = 1 - p
        have_next = g + 1 < NGROUP

        @pl.when(have_next)
        def _():
            pltpu.async_copy(src_hbm.at[wid, g + 1], sidx.at[q], fsem_s.at[q])
            pltpu.async_copy(dst_hbm.at[wid, g + 1], didx.at[q], fsem_d.at[q])

        for b in range(NBUF):
            gather_wait(p, b)
            scatter_start(p, b)

        @pl.when(have_next)
        def _():
            pltpu.make_async_copy(src_hbm.at[wid, g + 1], sidx.at[q],
                                  fsem_s.at[q]).wait()
            pltpu.make_async_copy(dst_hbm.at[wid, g + 1], didx.at[q],
                                  fsem_d.at[q]).wait()

        for b in range(NBUF):
            scatter_wait(p, b)

            @pl.when(have_next)
            def _():
                gather(q, b)

        return 0

    lax.fori_loop(0, NGROUP, group_body, 0)
    plsc.subcore_barrier()

    @pl.when(s < NS - 1)
    def _():
        pltpu.sync_copy(agg_s.at[pl.ds(s * OPT, OPT)],
                        out_hbm.at[c, pl.ds(s * OPT, OPT)])

    @pl.when(s == NS - 1)
    def _():
        pltpu.sync_copy(agg_s.at[pl.ds(15 * OPT, OPT_LAST)],
                        out_hbm.at[c, pl.ds(15 * OPT, OPT_LAST)])


BLK = 1000


def _tc_body(x_ref, a0_ref, a1_ref, w1_ref, b1_ref, w2_ref, b2_ref,
             wl_ref, bl_ref, o_ref):
    xb = x_ref[...]
    h = xb + a0_ref[...] + a1_ref[...]
    h = jnp.dot(h, w1_ref[...], preferred_element_type=jnp.float32) + b1_ref[...]
    h = jnp.maximum(h, 0.0)
    h = jnp.dot(h, w2_ref[...], preferred_element_type=jnp.float32) + b2_ref[...]
    res = jnp.dot(xb, wl_ref[...], preferred_element_type=jnp.float32) + bl_ref[...]
    o_ref[...] = res + h


def _tc_mlp(x, a0, a1, W1, b1, W2, b2, Wl, bl):
    grid = (N_NODES // BLK,)
    row_spec = pl.BlockSpec((BLK, D), lambda i: (i, 0))
    w_spec = pl.BlockSpec((D, D), lambda i: (0, 0))
    b_spec = pl.BlockSpec((1, D), lambda i: (0, 0))
    return pl.pallas_call(
        _tc_body,
        grid=grid,
        in_specs=[row_spec, row_spec, row_spec,
                  w_spec, b_spec, w_spec, b_spec, w_spec, b_spec],
        out_specs=row_spec,
        out_shape=jax.ShapeDtypeStruct((N_NODES, D), jnp.float32),
    )(x, a0, a1, W1, b1, W2, b2, Wl, bl)


@jax.jit
def kernel(x, edge_index, W1, b1, W2, b2, Wl, bl):
    ei = edge_index.astype(jnp.int32)
    pad = E_PAD - N_EDGES
    src = jnp.concatenate([ei[0], jnp.zeros((pad,), jnp.int32)])
    sink_rows = SINK + jnp.arange(pad, dtype=jnp.int32) % (AGG_ROWS - SINK)
    dst = jnp.concatenate([ei[1], sink_rows])
    src = src.reshape(NW, NGROUP, NBUF, CHUNK)
    dst = dst.reshape(NW, NGROUP, NBUF, CHUNK)
    zeros = jnp.zeros((AGG_ROWS, D), jnp.float32)
    agg = _sc_agg(x, src, dst, zeros)
    return _tc_mlp(x, agg[0], agg[1], W1,
                   b1.reshape(1, D), W2, b2.reshape(1, D),
                   Wl, bl.reshape(1, D))


# speedup vs baseline: 1.7606x; 1.7593x over previous
import functools

import jax
import jax.numpy as jnp
from jax import lax
from jax.experimental import pallas as pl
from jax.experimental.pallas import tpu as pltpu
from jax.experimental.pallas import tpu_sc as plsc

N_NODES = 10000
N_EDGES = 320000
D = 128

NC = 2
NS = 16
NW = NC * NS

CHUNK = 48
SETB = 3
NGROUP = 70
NCHUNK = NGROUP * SETB
EPT = NCHUNK * CHUNK
E_PAD = EPT * NW
SINK = N_NODES
AGG_ROWS = 10048
OPT = 632
OPT_LAST = N_NODES - 15 * OPT
ZPT_LAST = AGG_ROWS - 15 * OPT

_mesh = plsc.VectorSubcoreMesh(core_axis_name="c", subcore_axis_name="s")


@functools.partial(
    pl.kernel,
    mesh=_mesh,
    out_type=jax.ShapeDtypeStruct((NC, N_NODES, D), jnp.float32),
    scratch_types=[
        pltpu.VMEM((3, SETB, CHUNK), jnp.int32),
        pltpu.VMEM((3, SETB, CHUNK), jnp.int32),
        pltpu.VMEM((2, SETB, CHUNK, D), jnp.float32),
        pltpu.VMEM_SHARED((AGG_ROWS, D), jnp.float32),
        pltpu.SemaphoreType.DMA((2, SETB)),
        pltpu.SemaphoreType.DMA((2, SETB)),
        pltpu.SemaphoreType.DMA((3,)),
        pltpu.SemaphoreType.DMA((3,)),
    ],
)
def _sc_agg(x_hbm, src_hbm, dst_hbm, zeros_hbm, out_hbm,
            sidx, didx, rows_v, agg_s, gsem, ssem, fsem_s, fsem_d):
    c = lax.axis_index("c")
    s = lax.axis_index("s")
    wid = c * NS + s

    @pl.when(s < NS - 1)
    def _():
        pltpu.sync_copy(zeros_hbm.at[pl.ds(s * OPT, OPT)],
                        agg_s.at[pl.ds(s * OPT, OPT)])

    @pl.when(s == NS - 1)
    def _():
        pltpu.sync_copy(zeros_hbm.at[pl.ds(15 * OPT, ZPT_LAST)],
                        agg_s.at[pl.ds(15 * OPT, ZPT_LAST)])

    plsc.subcore_barrier()

    def gather(g_slot, p, b):
        pltpu.async_copy(x_hbm.at[sidx.at[g_slot, b]], rows_v.at[p, b],
                         gsem.at[p, b])

    def gather_wait(g_slot, p, b):
        pltpu.make_async_copy(x_hbm.at[sidx.at[g_slot, b]], rows_v.at[p, b],
                              gsem.at[p, b]).wait()

    def scatter_start(g_slot, p, b):
        pltpu.async_copy(rows_v.at[p, b], agg_s.at[didx.at[g_slot, b]],
                         ssem.at[p, b], add=True)

    def scatter_wait(g_slot, p, b):
        pltpu.make_async_copy(rows_v.at[p, b], agg_s.at[didx.at[g_slot, b]],
                              ssem.at[p, b]).wait()

    def idx_fetch(g, slot):
        pltpu.async_copy(src_hbm.at[wid, g], sidx.at[slot], fsem_s.at[slot])
        pltpu.async_copy(dst_hbm.at[wid, g], didx.at[slot], fsem_d.at[slot])

    def idx_wait(g, slot):
        pltpu.make_async_copy(src_hbm.at[wid, g], sidx.at[slot],
                              fsem_s.at[slot]).wait()
        pltpu.make_async_copy(dst_hbm.at[wid, g], didx.at[slot],
                              fsem_d.at[slot]).wait()

    pltpu.sync_copy(src_hbm.at[wid, 0], sidx.at[0])
    pltpu.sync_copy(dst_hbm.at[wid, 0], didx.at[0])
    idx_fetch(1, 1)
    for b in range(SETB):
        gather(0, 0, b)

    def group_body(g, _):
        p = lax.rem(g, 2)
        q = 1 - p
        r = lax.rem(g, 3)
        r1 = lax.rem(g + 1, 3)
        r2 = lax.rem(g + 2, 3)

        for b in range(SETB):
            gather_wait(r, p, b)
            scatter_start(r, p, b)

        @pl.when(g > 0)
        def _():
            for b in range(SETB):
                scatter_wait(r2, q, b)

        @pl.when(g + 1 < NGROUP)
        def _():
            idx_wait(g + 1, r1)
            for b in range(SETB):
                gather(r1, q, b)

        @pl.when(g + 2 < NGROUP)
        def _():
            idx_fetch(g + 2, r2)

        return 0

    lax.fori_loop(0, NGROUP, group_body, 0)
    for b in range(SETB):
        scatter_wait(lax.rem(NGROUP - 1, 3), lax.rem(NGROUP - 1, 2), b)
    plsc.subcore_barrier()

    @pl.when(s < NS - 1)
    def _():
        pltpu.sync_copy(agg_s.at[pl.ds(s * OPT, OPT)],
                        out_hbm.at[c, pl.ds(s * OPT, OPT)])

    @pl.when(s == NS - 1)
    def _():
        pltpu.sync_copy(agg_s.at[pl.ds(15 * OPT, OPT_LAST)],
                        out_hbm.at[c, pl.ds(15 * OPT, OPT_LAST)])


BLK = 1000


def _tc_body(x_ref, a0_ref, a1_ref, w1_ref, b1_ref, w2_ref, b2_ref,
             wl_ref, bl_ref, o_ref):
    xb = x_ref[...]
    h = xb + a0_ref[...] + a1_ref[...]
    h = jnp.dot(h, w1_ref[...], preferred_element_type=jnp.float32) + b1_ref[...]
    h = jnp.maximum(h, 0.0)
    h = jnp.dot(h, w2_ref[...], preferred_element_type=jnp.float32) + b2_ref[...]
    res = jnp.dot(xb, wl_ref[...], preferred_element_type=jnp.float32) + bl_ref[...]
    o_ref[...] = res + h


def _tc_mlp(x, a0, a1, W1, b1, W2, b2, Wl, bl):
    grid = (N_NODES // BLK,)
    row_spec = pl.BlockSpec((BLK, D), lambda i: (i, 0))
    w_spec = pl.BlockSpec((D, D), lambda i: (0, 0))
    b_spec = pl.BlockSpec((1, D), lambda i: (0, 0))
    return pl.pallas_call(
        _tc_body,
        grid=grid,
        in_specs=[row_spec, row_spec, row_spec,
                  w_spec, b_spec, w_spec, b_spec, w_spec, b_spec],
        out_specs=row_spec,
        out_shape=jax.ShapeDtypeStruct((N_NODES, D), jnp.float32),
    )(x, a0, a1, W1, b1, W2, b2, Wl, bl)


@jax.jit
def kernel(x, edge_index, W1, b1, W2, b2, Wl, bl):
    ei = edge_index.astype(jnp.int32)
    pad = E_PAD - N_EDGES
    src = jnp.concatenate([ei[0], jnp.zeros((pad,), jnp.int32)])
    sink_rows = SINK + jnp.arange(pad, dtype=jnp.int32) % (AGG_ROWS - SINK)
    dst = jnp.concatenate([ei[1], sink_rows])
    src = src.reshape(NW, NGROUP, SETB, CHUNK)
    dst = dst.reshape(NW, NGROUP, SETB, CHUNK)
    zeros = jnp.zeros((AGG_ROWS, D), jnp.float32)
    agg = _sc_agg(x, src, dst, zeros)
    return _tc_mlp(x, agg[0], agg[1], W1,
                   b1.reshape(1, D), W2, b2.reshape(1, D),
                   Wl, bl.reshape(1, D))


# speedup vs baseline: 2.3789x; 1.3512x over previous
import functools

import jax
import jax.numpy as jnp
from jax import lax
from jax.experimental import pallas as pl
from jax.experimental.pallas import tpu as pltpu
from jax.experimental.pallas import tpu_sc as plsc

N_NODES = 10000
N_EDGES = 320000
D = 128

NC = 2
NS = 16
NW = NC * NS

CHUNK = 48
SETB = 3
NG0 = 47
NG1 = 92
TOTAL_GROUPS = NS * (NG0 + NG1)
E_PAD = TOTAL_GROUPS * SETB * CHUNK
SINK = N_NODES
AGG_ROWS = 10048
OPT = 632
OPT_LAST = N_NODES - 15 * OPT
ZPT_LAST = AGG_ROWS - 15 * OPT

_mesh = plsc.VectorSubcoreMesh(core_axis_name="c", subcore_axis_name="s")


@functools.partial(
    pl.kernel,
    mesh=_mesh,
    out_type=jax.ShapeDtypeStruct((NC, N_NODES, D), jnp.float32),
    scratch_types=[
        pltpu.VMEM((3, SETB, CHUNK), jnp.int32),
        pltpu.VMEM((3, SETB, CHUNK), jnp.int32),
        pltpu.VMEM((2, SETB, CHUNK, D), jnp.float32),
        pltpu.VMEM_SHARED((AGG_ROWS, D), jnp.float32),
        pltpu.SemaphoreType.DMA((2, SETB)),
        pltpu.SemaphoreType.DMA((2, SETB)),
        pltpu.SemaphoreType.DMA((3,)),
        pltpu.SemaphoreType.DMA((3,)),
    ],
)
def _sc_agg(x_hbm, src_hbm, dst_hbm, zeros_hbm, out_hbm,
            sidx, didx, rows_v, agg_s, gsem, ssem, fsem_s, fsem_d):
    c = lax.axis_index("c")
    s = lax.axis_index("s")
    ng = jnp.where(c == 0, NG0, NG1)
    gstart = c * (NS * NG0) + s * ng

    @pl.when(s < NS - 1)
    def _():
        pltpu.sync_copy(zeros_hbm.at[pl.ds(s * OPT, OPT)],
                        agg_s.at[pl.ds(s * OPT, OPT)])

    @pl.when(s == NS - 1)
    def _():
        pltpu.sync_copy(zeros_hbm.at[pl.ds(15 * OPT, ZPT_LAST)],
                        agg_s.at[pl.ds(15 * OPT, ZPT_LAST)])

    plsc.subcore_barrier()

    def gather(g_slot, p, b):
        pltpu.async_copy(x_hbm.at[sidx.at[g_slot, b]], rows_v.at[p, b],
                         gsem.at[p, b])

    def gather_wait(g_slot, p, b):
        pltpu.make_async_copy(x_hbm.at[sidx.at[g_slot, b]], rows_v.at[p, b],
                              gsem.at[p, b]).wait()

    def scatter_start(g_slot, p, b):
        pltpu.async_copy(rows_v.at[p, b], agg_s.at[didx.at[g_slot, b]],
                         ssem.at[p, b], add=True)

    def scatter_wait(g_slot, p, b):
        pltpu.make_async_copy(rows_v.at[p, b], agg_s.at[didx.at[g_slot, b]],
                              ssem.at[p, b]).wait()

    def idx_fetch(g, slot):
        pltpu.async_copy(src_hbm.at[gstart + g], sidx.at[slot],
                         fsem_s.at[slot])
        pltpu.async_copy(dst_hbm.at[gstart + g], didx.at[slot],
                         fsem_d.at[slot])

    def idx_wait(g, slot):
        pltpu.make_async_copy(src_hbm.at[gstart + g], sidx.at[slot],
                              fsem_s.at[slot]).wait()
        pltpu.make_async_copy(dst_hbm.at[gstart + g], didx.at[slot],
                              fsem_d.at[slot]).wait()

    pltpu.sync_copy(src_hbm.at[gstart], sidx.at[0])
    pltpu.sync_copy(dst_hbm.at[gstart], didx.at[0])
    idx_fetch(1, 1)
    for b in range(SETB):
        gather(0, 0, b)

    def group_body(g, _):
        p = lax.rem(g, 2)
        q = 1 - p
        r = lax.rem(g, 3)
        r1 = lax.rem(g + 1, 3)
        r2 = lax.rem(g + 2, 3)

        for b in range(SETB):
            gather_wait(r, p, b)
            scatter_start(r, p, b)

        @pl.when(g > 0)
        def _():
            for b in range(SETB):
                scatter_wait(r2, q, b)

        @pl.when(g + 1 < ng)
        def _():
            idx_wait(g + 1, r1)
            for b in range(SETB):
                gather(r1, q, b)

        @pl.when(g + 2 < ng)
        def _():
            idx_fetch(g + 2, r2)

        return 0

    lax.fori_loop(0, ng, group_body, 0)
    for b in range(SETB):
        scatter_wait(lax.rem(ng - 1, 3), lax.rem(ng - 1, 2), b)
    plsc.subcore_barrier()

    @pl.when(s < NS - 1)
    def _():
        pltpu.sync_copy(agg_s.at[pl.ds(s * OPT, OPT)],
                        out_hbm.at[c, pl.ds(s * OPT, OPT)])

    @pl.when(s == NS - 1)
    def _():
        pltpu.sync_copy(agg_s.at[pl.ds(15 * OPT, OPT_LAST)],
                        out_hbm.at[c, pl.ds(15 * OPT, OPT_LAST)])


BLK = 1000


def _tc_body(x_ref, a0_ref, a1_ref, w1_ref, b1_ref, w2_ref, b2_ref,
             wl_ref, bl_ref, o_ref):
    xb = x_ref[...]
    h = xb + a0_ref[...] + a1_ref[...]
    h = jnp.dot(h, w1_ref[...], preferred_element_type=jnp.float32) + b1_ref[...]
    h = jnp.maximum(h, 0.0)
    h = jnp.dot(h, w2_ref[...], preferred_element_type=jnp.float32) + b2_ref[...]
    res = jnp.dot(xb, wl_ref[...], preferred_element_type=jnp.float32) + bl_ref[...]
    o_ref[...] = res + h


def _tc_mlp(x, a0, a1, W1, b1, W2, b2, Wl, bl):
    grid = (N_NODES // BLK,)
    row_spec = pl.BlockSpec((BLK, D), lambda i: (i, 0))
    w_spec = pl.BlockSpec((D, D), lambda i: (0, 0))
    b_spec = pl.BlockSpec((1, D), lambda i: (0, 0))
    return pl.pallas_call(
        _tc_body,
        grid=grid,
        in_specs=[row_spec, row_spec, row_spec,
                  w_spec, b_spec, w_spec, b_spec, w_spec, b_spec],
        out_specs=row_spec,
        out_shape=jax.ShapeDtypeStruct((N_NODES, D), jnp.float32),
    )(x, a0, a1, W1, b1, W2, b2, Wl, bl)


@jax.jit
def kernel(x, edge_index, W1, b1, W2, b2, Wl, bl):
    ei = edge_index.astype(jnp.int32)
    pad = E_PAD - N_EDGES
    src = jnp.concatenate([ei[0], jnp.zeros((pad,), jnp.int32)])
    sink_rows = SINK + jnp.arange(pad, dtype=jnp.int32) % (AGG_ROWS - SINK)
    dst = jnp.concatenate([ei[1], sink_rows])
    src = src.reshape(TOTAL_GROUPS, SETB, CHUNK)
    dst = dst.reshape(TOTAL_GROUPS, SETB, CHUNK)
    zeros = jnp.zeros((AGG_ROWS, D), jnp.float32)
    agg = _sc_agg(x, src, dst, zeros)
    return _tc_mlp(x, agg[0], agg[1], W1,
                   b1.reshape(1, D), W2, b2.reshape(1, D),
                   Wl, bl.reshape(1, D))


# speedup vs baseline: 2.7766x; 1.1672x over previous
import functools

import jax
import jax.numpy as jnp
from jax import lax
from jax.experimental import pallas as pl
from jax.experimental.pallas import tpu as pltpu
from jax.experimental.pallas import tpu_sc as plsc

N_NODES = 10000
N_EDGES = 320000
D = 128

NC = 2
NS = 16
NW = NC * NS

CHUNK = 48
SETB = 3
NG0 = 70
NG1 = 69
TOTAL_GROUPS = NS * (NG0 + NG1)
E_PAD = TOTAL_GROUPS * SETB * CHUNK
SINK = N_NODES
AGG_ROWS = 10048
OPT = 632
OPT_LAST = N_NODES - 15 * OPT
ZPT_LAST = AGG_ROWS - 15 * OPT

_mesh = plsc.VectorSubcoreMesh(core_axis_name="c", subcore_axis_name="s")


@functools.partial(
    pl.kernel,
    mesh=_mesh,
    out_type=jax.ShapeDtypeStruct((NC, N_NODES, D), jnp.float32),
    scratch_types=[
        pltpu.VMEM((3, SETB, CHUNK), jnp.int32),
        pltpu.VMEM((3, SETB, CHUNK), jnp.int32),
        pltpu.VMEM((2, SETB, CHUNK, D), jnp.float32),
        pltpu.VMEM_SHARED((AGG_ROWS, D), jnp.float32),
        pltpu.SemaphoreType.DMA((2, SETB)),
        pltpu.SemaphoreType.DMA((2, SETB)),
        pltpu.SemaphoreType.DMA((3,)),
        pltpu.SemaphoreType.DMA((3,)),
    ],
)
def _sc_agg(x_hbm, src_hbm, dst_hbm, zeros_hbm, out_hbm,
            sidx, didx, rows_v, agg_s, gsem, ssem, fsem_s, fsem_d):
    c = lax.axis_index("c")
    s = lax.axis_index("s")
    ng = jnp.where(c == 0, NG0, NG1)
    gstart = c * (NS * NG0) + s * ng

    @pl.when(s < NS - 1)
    def _():
        pltpu.sync_copy(zeros_hbm.at[pl.ds(s * OPT, OPT)],
                        agg_s.at[pl.ds(s * OPT, OPT)])

    @pl.when(s == NS - 1)
    def _():
        pltpu.sync_copy(zeros_hbm.at[pl.ds(15 * OPT, ZPT_LAST)],
                        agg_s.at[pl.ds(15 * OPT, ZPT_LAST)])

    plsc.subcore_barrier()

    def gather(g_slot, p, b):
        pltpu.async_copy(x_hbm.at[sidx.at[g_slot, b]], rows_v.at[p, b],
                         gsem.at[p, b])

    def gather_wait(g_slot, p, b):
        pltpu.make_async_copy(x_hbm.at[sidx.at[g_slot, b]], rows_v.at[p, b],
                              gsem.at[p, b]).wait()

    def scatter_start(g_slot, p, b):
        pltpu.async_copy(rows_v.at[p, b], agg_s.at[didx.at[g_slot, b]],
                         ssem.at[p, b], add=True)

    def scatter_wait(g_slot, p, b):
        pltpu.make_async_copy(rows_v.at[p, b], agg_s.at[didx.at[g_slot, b]],
                              ssem.at[p, b]).wait()

    def idx_fetch(g, slot):
        pltpu.async_copy(src_hbm.at[gstart + g], sidx.at[slot],
                         fsem_s.at[slot])
        pltpu.async_copy(dst_hbm.at[gstart + g], didx.at[slot],
                         fsem_d.at[slot])

    def idx_wait(g, slot):
        pltpu.make_async_copy(src_hbm.at[gstart + g], sidx.at[slot],
                              fsem_s.at[slot]).wait()
        pltpu.make_async_copy(dst_hbm.at[gstart + g], didx.at[slot],
                              fsem_d.at[slot]).wait()

    pltpu.sync_copy(src_hbm.at[gstart], sidx.at[0])
    pltpu.sync_copy(dst_hbm.at[gstart], didx.at[0])
    idx_fetch(1, 1)
    for b in range(SETB):
        gather(0, 0, b)

    def group_body(g, _):
        p = lax.rem(g, 2)
        q = 1 - p
        r = lax.rem(g, 3)
        r1 = lax.rem(g + 1, 3)
        r2 = lax.rem(g + 2, 3)

        for b in range(SETB):
            gather_wait(r, p, b)
            scatter_start(r, p, b)

        @pl.when(g > 0)
        def _():
            for b in range(SETB):
                scatter_wait(r2, q, b)

        @pl.when(g + 1 < ng)
        def _():
            idx_wait(g + 1, r1)
            for b in range(SETB):
                gather(r1, q, b)

        @pl.when(g + 2 < ng)
        def _():
            idx_fetch(g + 2, r2)

        return 0

    lax.fori_loop(0, ng, group_body, 0)
    for b in range(SETB):
        scatter_wait(lax.rem(ng - 1, 3), lax.rem(ng - 1, 2), b)
    plsc.subcore_barrier()

    @pl.when(s < NS - 1)
    def _():
        pltpu.sync_copy(agg_s.at[pl.ds(s * OPT, OPT)],
                        out_hbm.at[c, pl.ds(s * OPT, OPT)])

    @pl.when(s == NS - 1)
    def _():
        pltpu.sync_copy(agg_s.at[pl.ds(15 * OPT, OPT_LAST)],
                        out_hbm.at[c, pl.ds(15 * OPT, OPT_LAST)])


BLK = 1000


def _tc_body(x_ref, a0_ref, a1_ref, w1_ref, b1_ref, w2_ref, b2_ref,
             wl_ref, bl_ref, o_ref):
    xb = x_ref[...]
    h = xb + a0_ref[...] + a1_ref[...]
    h = jnp.dot(h, w1_ref[...], preferred_element_type=jnp.float32) + b1_ref[...]
    h = jnp.maximum(h, 0.0)
    h = jnp.dot(h, w2_ref[...], preferred_element_type=jnp.float32) + b2_ref[...]
    res = jnp.dot(xb, wl_ref[...], preferred_element_type=jnp.float32) + bl_ref[...]
    o_ref[...] = res + h


def _tc_mlp(x, a0, a1, W1, b1, W2, b2, Wl, bl):
    grid = (N_NODES // BLK,)
    row_spec = pl.BlockSpec((BLK, D), lambda i: (i, 0))
    w_spec = pl.BlockSpec((D, D), lambda i: (0, 0))
    b_spec = pl.BlockSpec((1, D), lambda i: (0, 0))
    return pl.pallas_call(
        _tc_body,
        grid=grid,
        in_specs=[row_spec, row_spec, row_spec,
                  w_spec, b_spec, w_spec, b_spec, w_spec, b_spec],
        out_specs=row_spec,
        out_shape=jax.ShapeDtypeStruct((N_NODES, D), jnp.float32),
    )(x, a0, a1, W1, b1, W2, b2, Wl, bl)


@jax.jit
def kernel(x, edge_index, W1, b1, W2, b2, Wl, bl):
    ei = edge_index.astype(jnp.int32)
    pad = E_PAD - N_EDGES
    src = jnp.concatenate([ei[0], jnp.zeros((pad,), jnp.int32)])
    sink_rows = SINK + jnp.arange(pad, dtype=jnp.int32) % (AGG_ROWS - SINK)
    dst = jnp.concatenate([ei[1], sink_rows])
    src = src.reshape(TOTAL_GROUPS, SETB, CHUNK)
    dst = dst.reshape(TOTAL_GROUPS, SETB, CHUNK)
    zeros = jnp.zeros((AGG_ROWS, D), jnp.float32)
    agg = _sc_agg(x, src, dst, zeros)
    return _tc_mlp(x, agg[0], agg[1], W1,
                   b1.reshape(1, D), W2, b2.reshape(1, D),
                   Wl, bl.reshape(1, D))


# speedup vs baseline: 3.0717x; 1.1063x over previous
import functools

import jax
import jax.numpy as jnp
from jax import lax
from jax.experimental import pallas as pl
from jax.experimental.pallas import tpu as pltpu
from jax.experimental.pallas import tpu_sc as plsc

N_NODES = 10000
N_EDGES = 320000
D = 128

NC = 2
NS = 16
NW = NC * NS

GEDGES = 128
SETB = 1
CHUNK = GEDGES // SETB
NGTOT = N_EDGES // GEDGES
NG_BASE = NGTOT // NW
NG_EXTRA = NGTOT - NG_BASE * NW
AGG_ROWS = 10016
OPT = 632
OPT_LAST = N_NODES - 15 * OPT
ZPT_LAST = AGG_ROWS - 15 * OPT

_mesh = plsc.VectorSubcoreMesh(core_axis_name="c", subcore_axis_name="s")


@functools.partial(
    pl.kernel,
    mesh=_mesh,
    out_type=jax.ShapeDtypeStruct((NC, N_NODES, D), jnp.float32),
    scratch_types=[
        pltpu.VMEM((3, CHUNK), jnp.int32),
        pltpu.VMEM((3, CHUNK), jnp.int32),
        pltpu.VMEM((2, SETB, CHUNK, D), jnp.float32),
        pltpu.VMEM_SHARED((AGG_ROWS, D), jnp.float32),
        pltpu.SemaphoreType.DMA((2, SETB)),
        pltpu.SemaphoreType.DMA((2, SETB)),
        pltpu.SemaphoreType.DMA((3,)),
        pltpu.SemaphoreType.DMA((3,)),
    ],
)
def _sc_agg(x_hbm, src_hbm, dst_hbm, zeros_hbm, out_hbm,
            sidx, didx, rows_v, agg_s, gsem, ssem, fsem_s, fsem_d):
    c = lax.axis_index("c")
    s = lax.axis_index("s")
    wid = c * NS + s
    ng = jnp.where(wid < NG_EXTRA, NG_BASE + 1, NG_BASE)
    gstart = NG_BASE * wid + jnp.minimum(wid, NG_EXTRA)

    @pl.when(s < NS - 1)
    def _():
        pltpu.sync_copy(zeros_hbm.at[pl.ds(s * OPT, OPT)],
                        agg_s.at[pl.ds(s * OPT, OPT)])

    @pl.when(s == NS - 1)
    def _():
        pltpu.sync_copy(zeros_hbm.at[pl.ds(15 * OPT, ZPT_LAST)],
                        agg_s.at[pl.ds(15 * OPT, ZPT_LAST)])

    plsc.subcore_barrier()

    def gather(g_slot, p, b):
        pltpu.async_copy(x_hbm.at[sidx.at[g_slot]], rows_v.at[p, b],
                         gsem.at[p, b])

    def gather_wait(g_slot, p, b):
        pltpu.make_async_copy(x_hbm.at[sidx.at[g_slot]], rows_v.at[p, b],
                              gsem.at[p, b]).wait()

    def scatter_start(g_slot, p, b):
        pltpu.async_copy(rows_v.at[p, b], agg_s.at[didx.at[g_slot]],
                         ssem.at[p, b], add=True)

    def scatter_wait(g_slot, p, b):
        pltpu.make_async_copy(rows_v.at[p, b], agg_s.at[didx.at[g_slot]],
                              ssem.at[p, b]).wait()

    def idx_fetch(g, slot):
        pltpu.async_copy(src_hbm.at[gstart + g], sidx.at[slot],
                         fsem_s.at[slot])
        pltpu.async_copy(dst_hbm.at[gstart + g], didx.at[slot],
                         fsem_d.at[slot])

    def idx_wait(g, slot):
        pltpu.make_async_copy(src_hbm.at[gstart + g], sidx.at[slot],
                              fsem_s.at[slot]).wait()
        pltpu.make_async_copy(dst_hbm.at[gstart + g], didx.at[slot],
                              fsem_d.at[slot]).wait()

    pltpu.sync_copy(src_hbm.at[gstart], sidx.at[0])
    pltpu.sync_copy(dst_hbm.at[gstart], didx.at[0])
    idx_fetch(1, 1)
    for b in range(SETB):
        gather(0, 0, b)

    def group_body(g, _):
        p = lax.rem(g, 2)
        q = 1 - p
        r = lax.rem(g, 3)
        r1 = lax.rem(g + 1, 3)
        r2 = lax.rem(g + 2, 3)

        for b in range(SETB):
            gather_wait(r, p, b)
            scatter_start(r, p, b)

        @pl.when(g > 0)
        def _():
            for b in range(SETB):
                scatter_wait(r2, q, b)

        @pl.when(g + 1 < ng)
        def _():
            idx_wait(g + 1, r1)
            for b in range(SETB):
                gather(r1, q, b)

        @pl.when(g + 2 < ng)
        def _():
            idx_fetch(g + 2, r2)

        return 0

    lax.fori_loop(0, ng, group_body, 0)
    for b in range(SETB):
        scatter_wait(lax.rem(ng - 1, 3), lax.rem(ng - 1, 2), b)
    plsc.subcore_barrier()

    @pl.when(s < NS - 1)
    def _():
        pltpu.sync_copy(agg_s.at[pl.ds(s * OPT, OPT)],
                        out_hbm.at[c, pl.ds(s * OPT, OPT)])

    @pl.when(s == NS - 1)
    def _():
        pltpu.sync_copy(agg_s.at[pl.ds(15 * OPT, OPT_LAST)],
                        out_hbm.at[c, pl.ds(15 * OPT, OPT_LAST)])


BLK = 1000


def _tc_body(x_ref, a0_ref, a1_ref, w1_ref, b1_ref, w2_ref, b2_ref,
             wl_ref, bl_ref, o_ref):
    xb = x_ref[...]
    h = xb + a0_ref[0] + a1_ref[0]
    h = jnp.dot(h, w1_ref[...], preferred_element_type=jnp.float32) + b1_ref[...]
    h = jnp.maximum(h, 0.0)
    h = jnp.dot(h, w2_ref[...], preferred_element_type=jnp.float32) + b2_ref[...]
    res = jnp.dot(xb, wl_ref[...], preferred_element_type=jnp.float32) + bl_ref[...]
    o_ref[...] = res + h


def _tc_mlp(x, agg, W1, b1, W2, b2, Wl, bl):
    grid = (N_NODES // BLK,)
    row_spec = pl.BlockSpec((BLK, D), lambda i: (i, 0))
    a0_spec = pl.BlockSpec((1, BLK, D), lambda i: (0, i, 0))
    a1_spec = pl.BlockSpec((1, BLK, D), lambda i: (1, i, 0))
    w_spec = pl.BlockSpec((D, D), lambda i: (0, 0))
    b_spec = pl.BlockSpec((1, D), lambda i: (0, 0))
    return pl.pallas_call(
        _tc_body,
        grid=grid,
        in_specs=[row_spec, a0_spec, a1_spec,
                  w_spec, b_spec, w_spec, b_spec, w_spec, b_spec],
        out_specs=row_spec,
        out_shape=jax.ShapeDtypeStruct((N_NODES, D), jnp.float32),
    )(x, agg, agg, W1, b1, W2, b2, Wl, bl)


@jax.jit
def kernel(x, edge_index, W1, b1, W2, b2, Wl, bl):
    ei = edge_index.astype(jnp.int32)
    src = ei[0].reshape(NGTOT, GEDGES)
    dst = ei[1].reshape(NGTOT, GEDGES)
    zeros = jnp.zeros((AGG_ROWS, D), jnp.float32)
    agg = _sc_agg(x, src, dst, zeros)
    return _tc_mlp(x, agg, W1,
                   b1.reshape(1, D), W2, b2.reshape(1, D),
                   Wl, bl.reshape(1, D))


# speedup vs baseline: 3.7569x; 1.2231x over previous
import functools

import jax
import jax.numpy as jnp
from jax import lax
from jax.experimental import pallas as pl
from jax.experimental.pallas import tpu as pltpu
from jax.experimental.pallas import tpu_sc as plsc

N_NODES = 10000
N_EDGES = 320000
D = 128

NC = 2
NS = 16
NW = NC * NS

GEDGES = 128
NSET = 3
CHUNK = GEDGES
NGTOT = N_EDGES // GEDGES
NG_BASE = NGTOT // NW
NG_EXTRA = NGTOT - NG_BASE * NW
AGG_ROWS = 10000
OPT = 632
OPT_LAST = N_NODES - 15 * OPT
ZPT_LAST = AGG_ROWS - 15 * OPT

_mesh = plsc.VectorSubcoreMesh(core_axis_name="c", subcore_axis_name="s")


@functools.partial(
    pl.kernel,
    mesh=_mesh,
    out_type=jax.ShapeDtypeStruct((NC, N_NODES, D), jnp.float32),
    scratch_types=[
        pltpu.VMEM((4, CHUNK), jnp.int32),
        pltpu.VMEM((4, CHUNK), jnp.int32),
        pltpu.VMEM((NSET, CHUNK, D), jnp.float32),
        pltpu.VMEM_SHARED((AGG_ROWS, D), jnp.float32),
        pltpu.SemaphoreType.DMA((NSET,)),
        pltpu.SemaphoreType.DMA((NSET,)),
        pltpu.SemaphoreType.DMA((4,)),
        pltpu.SemaphoreType.DMA((4,)),
    ],
)
def _sc_agg(x_hbm, src_hbm, dst_hbm, zeros_hbm, out_hbm,
            sidx, didx, rows_v, agg_s, gsem, ssem, fsem_s, fsem_d):
    c = lax.axis_index("c")
    s = lax.axis_index("s")
    wid = c * NS + s
    ng = jnp.where(wid < NG_EXTRA, NG_BASE + 1, NG_BASE)
    gstart = NG_BASE * wid + jnp.minimum(wid, NG_EXTRA)

    @pl.when(s < NS - 1)
    def _():
        pltpu.sync_copy(zeros_hbm.at[pl.ds(s * OPT, OPT)],
                        agg_s.at[pl.ds(s * OPT, OPT)])

    @pl.when(s == NS - 1)
    def _():
        pltpu.sync_copy(zeros_hbm.at[pl.ds(15 * OPT, ZPT_LAST)],
                        agg_s.at[pl.ds(15 * OPT, ZPT_LAST)])

    plsc.subcore_barrier()

    def gather(g_slot, m):
        pltpu.async_copy(x_hbm.at[sidx.at[g_slot]], rows_v.at[m],
                         gsem.at[m])

    def gather_wait(g_slot, m):
        pltpu.make_async_copy(x_hbm.at[sidx.at[g_slot]], rows_v.at[m],
                              gsem.at[m]).wait()

    def scatter_start(g_slot, m):
        pltpu.async_copy(rows_v.at[m], agg_s.at[didx.at[g_slot]],
                         ssem.at[m], add=True)

    def scatter_wait(g_slot, m):
        pltpu.make_async_copy(rows_v.at[m], agg_s.at[didx.at[g_slot]],
                              ssem.at[m]).wait()

    def idx_fetch(g, slot):
        pltpu.async_copy(src_hbm.at[gstart + g], sidx.at[slot],
                         fsem_s.at[slot])
        pltpu.async_copy(dst_hbm.at[gstart + g], didx.at[slot],
                         fsem_d.at[slot])

    def idx_wait(g, slot):
        pltpu.make_async_copy(src_hbm.at[gstart + g], sidx.at[slot],
                              fsem_s.at[slot]).wait()
        pltpu.make_async_copy(dst_hbm.at[gstart + g], didx.at[slot],
                              fsem_d.at[slot]).wait()

    pltpu.sync_copy(src_hbm.at[gstart], sidx.at[0])
    pltpu.sync_copy(dst_hbm.at[gstart], didx.at[0])
    idx_fetch(1, 1)
    idx_fetch(2, 2)
    gather(0, 0)
    idx_wait(1, 1)
    gather(1, 1)

    def group_body(g, _):
        m = lax.rem(g, NSET)
        m2 = lax.rem(g + 2, NSET)
        sg = lax.rem(g, 4)
        sm1 = lax.rem(g + 3, 4)
        s2 = lax.rem(g + 2, 4)
        s3 = lax.rem(g + 3, 4)

        gather_wait(sg, m)
        scatter_start(sg, m)

        @pl.when(g > 0)
        def _():
            scatter_wait(sm1, m2)

        @pl.when(g + 2 < ng)
        def _():
            idx_wait(g + 2, s2)
            gather(s2, m2)

        @pl.when(g + 3 < ng)
        def _():
            idx_fetch(g + 3, s3)

        return 0

    lax.fori_loop(0, ng, group_body, 0)
    scatter_wait(lax.rem(ng - 1, 4), lax.rem(ng - 1, NSET))
    plsc.subcore_barrier()

    @pl.when(s < NS - 1)
    def _():
        pltpu.sync_copy(agg_s.at[pl.ds(s * OPT, OPT)],
                        out_hbm.at[c, pl.ds(s * OPT, OPT)])

    @pl.when(s == NS - 1)
    def _():
        pltpu.sync_copy(agg_s.at[pl.ds(15 * OPT, OPT_LAST)],
                        out_hbm.at[c, pl.ds(15 * OPT, OPT_LAST)])


BLK = 1000


def _tc_body(x_ref, a0_ref, a1_ref, w1_ref, b1_ref, w2_ref, b2_ref,
             wl_ref, bl_ref, o_ref):
    xb = x_ref[...]
    h = xb + a0_ref[0] + a1_ref[0]
    h = jnp.dot(h, w1_ref[...], preferred_element_type=jnp.float32) + b1_ref[...]
    h = jnp.maximum(h, 0.0)
    h = jnp.dot(h, w2_ref[...], preferred_element_type=jnp.float32) + b2_ref[...]
    res = jnp.dot(xb, wl_ref[...], preferred_element_type=jnp.float32) + bl_ref[...]
    o_ref[...] = res + h


def _tc_mlp(x, agg, W1, b1, W2, b2, Wl, bl):
    grid = (N_NODES // BLK,)
    row_spec = pl.BlockSpec((BLK, D), lambda i: (i, 0))
    a0_spec = pl.BlockSpec((1, BLK, D), lambda i: (0, i, 0))
    a1_spec = pl.BlockSpec((1, BLK, D), lambda i: (1, i, 0))
    w_spec = pl.BlockSpec((D, D), lambda i: (0, 0))
    b_spec = pl.BlockSpec((1, D), lambda i: (0, 0))
    return pl.pallas_call(
        _tc_body,
        grid=grid,
        in_specs=[row_spec, a0_spec, a1_spec,
                  w_spec, b_spec, w_spec, b_spec, w_spec, b_spec],
        out_specs=row_spec,
        out_shape=jax.ShapeDtypeStruct((N_NODES, D), jnp.float32),
    )(x, agg, agg, W1, b1, W2, b2, Wl, bl)


@jax.jit
def kernel(x, edge_index, W1, b1, W2, b2, Wl, bl):
    ei = edge_index.astype(jnp.int32)
    src = ei[0].reshape(NGTOT, GEDGES)
    dst = ei[1].reshape(NGTOT, GEDGES)
    zeros = jnp.zeros((AGG_ROWS, D), jnp.float32)
    agg = _sc_agg(x, src, dst, zeros)
    return _tc_mlp(x, agg, W1,
                   b1.reshape(1, D), W2, b2.reshape(1, D),
                   Wl, bl.reshape(1, D))


# speedup vs baseline: 4.1966x; 1.1170x over previous
import functools

import jax
import jax.numpy as jnp
from jax import lax
from jax.experimental import pallas as pl
from jax.experimental.pallas import tpu as pltpu
from jax.experimental.pallas import tpu_sc as plsc

N_NODES = 10000
N_EDGES = 320000
D = 128

NC = 2
NS = 16
NW = NC * NS

GEDGES = 128
NSET = 3
CHUNK = GEDGES
NGTOT = N_EDGES // GEDGES
NG_BASE = NGTOT // NW
NG_EXTRA = NGTOT - NG_BASE * NW
AGG_ROWS = 10000
OPT = 632
OPT_LAST = N_NODES - 15 * OPT
ZPT_LAST = AGG_ROWS - 15 * OPT

_mesh = plsc.VectorSubcoreMesh(core_axis_name="c", subcore_axis_name="s")


@functools.partial(
    pl.kernel,
    mesh=_mesh,
    out_type=jax.ShapeDtypeStruct((NC, N_NODES, D), jnp.float32),
    scratch_types=[
        pltpu.VMEM((4, CHUNK), jnp.int32),
        pltpu.VMEM((4, CHUNK), jnp.int32),
        pltpu.VMEM((NSET, CHUNK, D), jnp.float32),
        pltpu.VMEM_SHARED((AGG_ROWS, D), jnp.float32),
        pltpu.SemaphoreType.DMA((NSET,)),
        pltpu.SemaphoreType.DMA((NSET,)),
        pltpu.SemaphoreType.DMA((4,)),
        pltpu.SemaphoreType.DMA((4,)),
    ],
)
def _sc_agg(x_hbm, ei_hbm, zeros_hbm, out_hbm,
            sidx, didx, rows_v, agg_s, gsem, ssem, fsem_s, fsem_d):
    c = lax.axis_index("c")
    s = lax.axis_index("s")
    wid = c * NS + s
    ng = jnp.where(wid < NG_EXTRA, NG_BASE + 1, NG_BASE)
    gstart = NG_BASE * wid + jnp.minimum(wid, NG_EXTRA)

    @pl.when(s < NS - 1)
    def _():
        pltpu.sync_copy(zeros_hbm, agg_s.at[pl.ds(s * OPT, OPT)])

    @pl.when(s == NS - 1)
    def _():
        pltpu.sync_copy(zeros_hbm.at[pl.ds(0, ZPT_LAST)],
                        agg_s.at[pl.ds(15 * OPT, ZPT_LAST)])

    plsc.subcore_barrier()

    def gather(g_slot, m):
        pltpu.async_copy(x_hbm.at[sidx.at[g_slot]], rows_v.at[m],
                         gsem.at[m])

    def gather_wait(g_slot, m):
        pltpu.make_async_copy(x_hbm.at[sidx.at[g_slot]], rows_v.at[m],
                              gsem.at[m]).wait()

    def scatter_start(g_slot, m):
        pltpu.async_copy(rows_v.at[m], agg_s.at[didx.at[g_slot]],
                         ssem.at[m], add=True)

    def scatter_wait(g_slot, m):
        pltpu.make_async_copy(rows_v.at[m], agg_s.at[didx.at[g_slot]],
                              ssem.at[m]).wait()

    def idx_fetch(g, slot):
        pltpu.async_copy(ei_hbm.at[0, gstart + g], sidx.at[slot],
                         fsem_s.at[slot])
        pltpu.async_copy(ei_hbm.at[1, gstart + g], didx.at[slot],
                         fsem_d.at[slot])

    def idx_wait(g, slot):
        pltpu.make_async_copy(ei_hbm.at[0, gstart + g], sidx.at[slot],
                              fsem_s.at[slot]).wait()
        pltpu.make_async_copy(ei_hbm.at[1, gstart + g], didx.at[slot],
                              fsem_d.at[slot]).wait()

    pltpu.sync_copy(ei_hbm.at[0, gstart], sidx.at[0])
    pltpu.sync_copy(ei_hbm.at[1, gstart], didx.at[0])
    idx_fetch(1, 1)
    idx_fetch(2, 2)
    gather(0, 0)
    idx_wait(1, 1)
    gather(1, 1)

    def group_body(g, _):
        m = lax.rem(g, NSET)
        m2 = lax.rem(g + 2, NSET)
        sg = lax.rem(g, 4)
        sm1 = lax.rem(g + 3, 4)
        s2 = lax.rem(g + 2, 4)
        s3 = lax.rem(g + 3, 4)

        gather_wait(sg, m)
        scatter_start(sg, m)

        @pl.when(g > 0)
        def _():
            scatter_wait(sm1, m2)

        @pl.when(g + 2 < ng)
        def _():
            idx_wait(g + 2, s2)
            gather(s2, m2)

        @pl.when(g + 3 < ng)
        def _():
            idx_fetch(g + 3, s3)

        return 0

    lax.fori_loop(0, ng, group_body, 0)
    scatter_wait(lax.rem(ng - 1, 4), lax.rem(ng - 1, NSET))
    plsc.subcore_barrier()

    @pl.when(s < NS - 1)
    def _():
        pltpu.sync_copy(agg_s.at[pl.ds(s * OPT, OPT)],
                        out_hbm.at[c, pl.ds(s * OPT, OPT)])

    @pl.when(s == NS - 1)
    def _():
        pltpu.sync_copy(agg_s.at[pl.ds(15 * OPT, OPT_LAST)],
                        out_hbm.at[c, pl.ds(15 * OPT, OPT_LAST)])


BLK = 2000


def _tc_body(x_ref, a0_ref, a1_ref, w1_ref, b1_ref, w2_ref, b2_ref,
             wl_ref, bl_ref, o_ref):
    xb = x_ref[...]
    h = xb + a0_ref[0] + a1_ref[0]
    h = jnp.dot(h, w1_ref[...], preferred_element_type=jnp.float32) + b1_ref[...]
    h = jnp.maximum(h, 0.0)
    h = jnp.dot(h, w2_ref[...], preferred_element_type=jnp.float32) + b2_ref[...]
    res = jnp.dot(xb, wl_ref[...], preferred_element_type=jnp.float32) + bl_ref[...]
    o_ref[...] = res + h


def _tc_mlp(x, agg, W1, b1, W2, b2, Wl, bl):
    grid = (N_NODES // BLK,)
    row_spec = pl.BlockSpec((BLK, D), lambda i: (i, 0))
    a0_spec = pl.BlockSpec((1, BLK, D), lambda i: (0, i, 0))
    a1_spec = pl.BlockSpec((1, BLK, D), lambda i: (1, i, 0))
    w_spec = pl.BlockSpec((D, D), lambda i: (0, 0))
    b_spec = pl.BlockSpec((1, D), lambda i: (0, 0))
    return pl.pallas_call(
        _tc_body,
        grid=grid,
        in_specs=[row_spec, a0_spec, a1_spec,
                  w_spec, b_spec, w_spec, b_spec, w_spec, b_spec],
        out_specs=row_spec,
        out_shape=jax.ShapeDtypeStruct((N_NODES, D), jnp.float32),
    )(x, agg, agg, W1, b1, W2, b2, Wl, bl)


@jax.jit
def kernel(x, edge_index, W1, b1, W2, b2, Wl, bl):
    ei3 = edge_index.astype(jnp.int32).reshape(2, NGTOT, GEDGES)
    zeros = jnp.zeros((OPT, D), jnp.float32)
    agg = _sc_agg(x, ei3, zeros)
    return _tc_mlp(x, agg, W1,
                   b1.reshape(1, D), W2, b2.reshape(1, D),
                   Wl, bl.reshape(1, D))


# speedup vs baseline: 4.2000x; 1.0008x over previous
import functools

import jax
import jax.numpy as jnp
from jax import lax
from jax.experimental import pallas as pl
from jax.experimental.pallas import tpu as pltpu
from jax.experimental.pallas import tpu_sc as plsc

N_NODES = 10000
N_EDGES = 320000
D = 128

NC = 2
NS = 16
NW = NC * NS

GEDGES = 128
NSET = 3
CHUNK = GEDGES
NGTOT = N_EDGES // GEDGES
NG_BASE = NGTOT // NW
NG_EXTRA = NGTOT - NG_BASE * NW
AGG_ROWS = 10000
OPT = 632
OPT_LAST = N_NODES - 15 * OPT
ZPT_LAST = AGG_ROWS - 15 * OPT

_mesh = plsc.VectorSubcoreMesh(core_axis_name="c", subcore_axis_name="s")


@functools.partial(
    pl.kernel,
    mesh=_mesh,
    out_type=jax.ShapeDtypeStruct((NC, N_NODES, D), jnp.float32),
    scratch_types=[
        pltpu.VMEM((4, CHUNK), jnp.int32),
        pltpu.VMEM((4, CHUNK), jnp.int32),
        pltpu.VMEM((NSET, CHUNK, D), jnp.float32),
        pltpu.VMEM_SHARED((AGG_ROWS, D), jnp.float32),
        pltpu.SemaphoreType.DMA((NSET,)),
        pltpu.SemaphoreType.DMA((NSET,)),
        pltpu.SemaphoreType.DMA((4,)),
        pltpu.SemaphoreType.DMA((4,)),
    ],
)
def _sc_agg(x_hbm, ei_hbm, zeros_hbm, out_hbm,
            sidx, didx, rows_v, agg_s, gsem, ssem, fsem_s, fsem_d):
    c = lax.axis_index("c")
    s = lax.axis_index("s")
    wid = c * NS + s
    ng = jnp.where(wid < NG_EXTRA, NG_BASE + 1, NG_BASE)
    gstart = NG_BASE * wid + jnp.minimum(wid, NG_EXTRA)

    def gather(g_slot, m):
        pltpu.async_copy(x_hbm.at[sidx.at[g_slot]], rows_v.at[m],
                         gsem.at[m])

    def gather_wait(g_slot, m):
        pltpu.make_async_copy(x_hbm.at[sidx.at[g_slot]], rows_v.at[m],
                              gsem.at[m]).wait()

    def scatter_start(g_slot, m):
        pltpu.async_copy(rows_v.at[m], agg_s.at[didx.at[g_slot]],
                         ssem.at[m], add=True)

    def scatter_wait(g_slot, m):
        pltpu.make_async_copy(rows_v.at[m], agg_s.at[didx.at[g_slot]],
                              ssem.at[m]).wait()

    def idx_fetch(g, slot):
        pltpu.async_copy(ei_hbm.at[0, gstart + g], sidx.at[slot],
                         fsem_s.at[slot])
        pltpu.async_copy(ei_hbm.at[1, gstart + g], didx.at[slot],
                         fsem_d.at[slot])

    def idx_wait(g, slot):
        pltpu.make_async_copy(ei_hbm.at[0, gstart + g], sidx.at[slot],
                              fsem_s.at[slot]).wait()
        pltpu.make_async_copy(ei_hbm.at[1, gstart + g], didx.at[slot],
                              fsem_d.at[slot]).wait()

    pltpu.sync_copy(ei_hbm.at[0, gstart], sidx.at[0])
    pltpu.sync_copy(ei_hbm.at[1, gstart], didx.at[0])
    idx_fetch(1, 1)
    idx_fetch(2, 2)
    gather(0, 0)

    @pl.when(s < NS - 1)
    def _():
        pltpu.sync_copy(zeros_hbm, agg_s.at[pl.ds(s * OPT, OPT)])

    @pl.when(s == NS - 1)
    def _():
        pltpu.sync_copy(zeros_hbm.at[pl.ds(0, ZPT_LAST)],
                        agg_s.at[pl.ds(15 * OPT, ZPT_LAST)])

    idx_wait(1, 1)
    gather(1, 1)
    plsc.subcore_barrier()

    def group_body(g, _):
        m = lax.rem(g, NSET)
        m2 = lax.rem(g + 2, NSET)
        sg = lax.rem(g, 4)
        sm1 = lax.rem(g + 3, 4)
        s2 = lax.rem(g + 2, 4)
        s3 = lax.rem(g + 3, 4)

        gather_wait(sg, m)
        scatter_start(sg, m)

        @pl.when(g > 0)
        def _():
            scatter_wait(sm1, m2)

        @pl.when(g + 2 < ng)
        def _():
            idx_wait(g + 2, s2)
            gather(s2, m2)

        @pl.when(g + 3 < ng)
        def _():
            idx_fetch(g + 3, s3)

        return 0

    lax.fori_loop(0, ng, group_body, 0)
    scatter_wait(lax.rem(ng - 1, 4), lax.rem(ng - 1, NSET))
    plsc.subcore_barrier()

    @pl.when(s < NS - 1)
    def _():
        pltpu.sync_copy(agg_s.at[pl.ds(s * OPT, OPT)],
                        out_hbm.at[c, pl.ds(s * OPT, OPT)])

    @pl.when(s == NS - 1)
    def _():
        pltpu.sync_copy(agg_s.at[pl.ds(15 * OPT, OPT_LAST)],
                        out_hbm.at[c, pl.ds(15 * OPT, OPT_LAST)])


BLK = 2000


def _tc_body(x_ref, a0_ref, a1_ref, w1_ref, b1_ref, w2_ref, b2_ref,
             wl_ref, bl_ref, o_ref):
    xb = x_ref[...]
    h = xb + a0_ref[0] + a1_ref[0]
    h = jnp.dot(h, w1_ref[...], preferred_element_type=jnp.float32) + b1_ref[...]
    h = jnp.maximum(h, 0.0)
    h = jnp.dot(h, w2_ref[...], preferred_element_type=jnp.float32) + b2_ref[...]
    res = jnp.dot(xb, wl_ref[...], preferred_element_type=jnp.float32) + bl_ref[...]
    o_ref[...] = res + h


def _tc_mlp(x, agg, W1, b1, W2, b2, Wl, bl):
    grid = (N_NODES // BLK,)
    row_spec = pl.BlockSpec((BLK, D), lambda i: (i, 0))
    a0_spec = pl.BlockSpec((1, BLK, D), lambda i: (0, i, 0))
    a1_spec = pl.BlockSpec((1, BLK, D), lambda i: (1, i, 0))
    w_spec = pl.BlockSpec((D, D), lambda i: (0, 0))
    b_spec = pl.BlockSpec((1, D), lambda i: (0, 0))
    return pl.pallas_call(
        _tc_body,
        grid=grid,
        in_specs=[row_spec, a0_spec, a1_spec,
                  w_spec, b_spec, w_spec, b_spec, w_spec, b_spec],
        out_specs=row_spec,
        out_shape=jax.ShapeDtypeStruct((N_NODES, D), jnp.float32),
    )(x, agg, agg, W1, b1, W2, b2, Wl, bl)


@jax.jit
def kernel(x, edge_index, W1, b1, W2, b2, Wl, bl):
    ei3 = edge_index.astype(jnp.int32).reshape(2, NGTOT, GEDGES)
    zeros = jnp.zeros((OPT, D), jnp.float32)
    agg = _sc_agg(x, ei3, zeros)
    return _tc_mlp(x, agg, W1,
                   b1.reshape(1, D), W2, b2.reshape(1, D),
                   Wl, bl.reshape(1, D))
